# Initial kernel scaffold; baseline (speedup 1.0000x reference)
import jax
import jax.numpy as jnp
from jax.experimental import pallas as pl


def kernel(rider_x, order_x, rmlp_W1, rmlp_b1, rmlp_W2, rmlp_b2, omlp_W1, omlp_b1, omlp_W2, omlp_b2, c1_ro_Wl, c1_ro_bl, c1_ro_Wr, c1_or_Wl, c1_or_bl, c1_or_Wr, c2_ro_Wl, c2_ro_bl, c2_ro_Wr, c2_or_Wl, c2_or_bl, c2_or_Wr, sc_W1, sc_b1, sc_W2, sc_b2, edge_index):
    raise NotImplementedError("write your pallas kernel here")


# speedup vs baseline: 2.7312x; 2.7312x over previous
import jax
import jax.numpy as jnp
from jax import lax
from jax.experimental import pallas as pl
from jax.experimental.pallas import tpu as pltpu
from jax.experimental.pallas import tpu_sc as plsc

N_R = 10000
N_O = 10000
E = 320000
D = 128
H = 64

NC = 2
NS = 16
NW = NC * NS
EPW = E // NW
CH = 80
NCHUNK = EPW // CH
NP = 10240
RPT = NP // NS

_MESH = plsc.VectorSubcoreMesh(
    core_axis_name="c", subcore_axis_name="s", num_cores=NC, num_subcores=NS)


def _tc_encode_body(rx, ox, rW1, rb1, rW2, rb2, oW1, ob1, oW2, ob2,
                    xr_out, xo_out):
    hr = jnp.maximum(
        jnp.dot(rx[...], rW1[...], preferred_element_type=jnp.float32)
        + rb1[...], 0.0)
    xr_out[...] = jnp.dot(hr, rW2[...],
                          preferred_element_type=jnp.float32) + rb2[...]
    ho = jnp.maximum(
        jnp.dot(ox[...], oW1[...], preferred_element_type=jnp.float32)
        + ob1[...], 0.0)
    xo_out[...] = jnp.dot(ho, oW2[...],
                          preferred_element_type=jnp.float32) + ob2[...]


def _mean(msum, cnt):
    m = msum[0] + msum[1]
    c = jnp.maximum(cnt[0, :, 0:1] + cnt[1, :, 0:1], 1.0)
    return m / c


def _tc_conv_body(mo_s, co_s, xo, Wl_o, bl_o, Wr_o,
                  mr_s, cr_s, xr, Wl_r, bl_r, Wr_r,
                  xo1_out, xr1_out):
    mo = _mean(mo_s, co_s)
    xo1_out[...] = jnp.maximum(
        jnp.dot(mo, Wl_o[...], preferred_element_type=jnp.float32) + bl_o[...]
        + jnp.dot(xo[...], Wr_o[...], preferred_element_type=jnp.float32), 0.0)
    mr = _mean(mr_s, cr_s)
    xr1_out[...] = jnp.maximum(
        jnp.dot(mr, Wl_r[...], preferred_element_type=jnp.float32) + bl_r[...]
        + jnp.dot(xr[...], Wr_r[...], preferred_element_type=jnp.float32), 0.0)


def _tc_final_body(mo_s, co_s, xo, Wl_o, bl_o, Wr_o, Wbot, b1,
                   mr_s, cr_s, xr, Wl_r, bl_r, Wr_r, Wtop,
                   q_out, p_out):
    mo = _mean(mo_s, co_s)
    xo2 = jnp.maximum(
        jnp.dot(mo, Wl_o[...], preferred_element_type=jnp.float32) + bl_o[...]
        + jnp.dot(xo[...], Wr_o[...], preferred_element_type=jnp.float32), 0.0)
    q_out[...] = jnp.dot(xo2, Wbot[...],
                         preferred_element_type=jnp.float32) + b1[...]
    mr = _mean(mr_s, cr_s)
    xr2 = jnp.maximum(
        jnp.dot(mr, Wl_r[...], preferred_element_type=jnp.float32) + bl_r[...]
        + jnp.dot(xr[...], Wr_r[...], preferred_element_type=jnp.float32), 0.0)
    p_out[...] = jnp.dot(xr2, Wtop[...], preferred_element_type=jnp.float32)


_f32 = jnp.float32

BR = 2000

def _bs_x(i):
    return pl.BlockSpec((BR, D), lambda i: (i, 0))

_BS_M = pl.BlockSpec((2, BR, H), lambda i: (0, i, 0))
_BS_C = pl.BlockSpec((2, BR, 16), lambda i: (0, i, 0))
_BS_N = pl.BlockSpec((BR, H), lambda i: (i, 0))
_BS_W = pl.BlockSpec((H, H), lambda i: (0, 0))
_BS_B = pl.BlockSpec((1, H), lambda i: (0, 0))
_BS_XD = pl.BlockSpec((BR, D), lambda i: (i, 0))
_BS_WD = pl.BlockSpec((D, H), lambda i: (0, 0))

_tc_encode = pl.pallas_call(
    _tc_encode_body,
    grid=(N_R // BR,),
    in_specs=[_BS_XD, _BS_XD, _BS_WD, _BS_B, _BS_W, _BS_B,
              _BS_WD, _BS_B, _BS_W, _BS_B],
    out_specs=[_BS_N, _BS_N],
    out_shape=[jax.ShapeDtypeStruct((N_R, H), _f32),
               jax.ShapeDtypeStruct((N_O, H), _f32)])

_tc_conv = pl.pallas_call(
    _tc_conv_body,
    grid=(N_O // BR,),
    in_specs=[_BS_M, _BS_C, _BS_N, _BS_W, _BS_B, _BS_W,
              _BS_M, _BS_C, _BS_N, _BS_W, _BS_B, _BS_W],
    out_specs=[_BS_N, _BS_N],
    out_shape=[jax.ShapeDtypeStruct((N_O, H), _f32),
               jax.ShapeDtypeStruct((N_R, H), _f32)])

_tc_final = pl.pallas_call(
    _tc_final_body,
    grid=(N_O // BR,),
    in_specs=[_BS_M, _BS_C, _BS_N, _BS_W, _BS_B, _BS_W, _BS_W, _BS_B,
              _BS_M, _BS_C, _BS_N, _BS_W, _BS_B, _BS_W, _BS_W],
    out_specs=[_BS_N, _BS_N],
    out_shape=[jax.ShapeDtypeStruct((N_O, H), _f32),
               jax.ShapeDtypeStruct((N_R, H), _f32)])


def _worker_id():
    return lax.axis_index("c") * NS + lax.axis_index("s")


def _sc_conv_cnt_body(xr_hbm, xo_hbm, src_hbm, dst_hbm, z64_hbm, z16_hbm,
                      ones_hbm,
                      mo_out, mr_out, co_out, cr_out,
                      src_v, dst_v, rowr_v, rowo_v, ones_v, sem1, sem2,
                      mo_sh, mr_sh, co_sh, cr_sh):
    c = lax.axis_index("c")
    s = lax.axis_index("s")
    wid = c * NS + s
    r0 = s * RPT
    pltpu.sync_copy(z64_hbm.at[pl.ds(r0, RPT)], mo_sh.at[pl.ds(r0, RPT)])
    pltpu.sync_copy(z64_hbm.at[pl.ds(r0, RPT)], mr_sh.at[pl.ds(r0, RPT)])
    pltpu.sync_copy(z16_hbm.at[pl.ds(r0, RPT)], co_sh.at[pl.ds(r0, RPT)])
    pltpu.sync_copy(z16_hbm.at[pl.ds(r0, RPT)], cr_sh.at[pl.ds(r0, RPT)])
    pltpu.sync_copy(ones_hbm, ones_v)
    plsc.subcore_barrier()

    def chunk(i, carry):
        base = wid * EPW + i * CH
        pltpu.sync_copy(src_hbm.at[pl.ds(base, CH)], src_v)
        pltpu.sync_copy(dst_hbm.at[pl.ds(base, CH)], dst_v)
        pltpu.async_copy(xr_hbm.at[src_v], rowr_v, sem1).wait()
        pltpu.async_copy(xo_hbm.at[dst_v], rowo_v, sem2).wait()
        pltpu.sync_copy(rowr_v, mo_sh.at[dst_v], add=True)
        pltpu.sync_copy(rowo_v, mr_sh.at[src_v], add=True)
        pltpu.sync_copy(ones_v, co_sh.at[dst_v], add=True)
        pltpu.sync_copy(ones_v, cr_sh.at[src_v], add=True)
        return carry

    lax.fori_loop(0, NCHUNK, chunk, 0)
    plsc.subcore_barrier()
    pltpu.sync_copy(mo_sh.at[pl.ds(r0, RPT)], mo_out.at[c, pl.ds(r0, RPT)])
    pltpu.sync_copy(mr_sh.at[pl.ds(r0, RPT)], mr_out.at[c, pl.ds(r0, RPT)])
    pltpu.sync_copy(co_sh.at[pl.ds(r0, RPT)], co_out.at[c, pl.ds(r0, RPT)])
    pltpu.sync_copy(cr_sh.at[pl.ds(r0, RPT)], cr_out.at[c, pl.ds(r0, RPT)])


_sc_conv_cnt = pl.kernel(
    _sc_conv_cnt_body,
    out_type=[jax.ShapeDtypeStruct((NC, NP, H), _f32),
              jax.ShapeDtypeStruct((NC, NP, H), _f32),
              jax.ShapeDtypeStruct((NC, NP, 16), _f32),
              jax.ShapeDtypeStruct((NC, NP, 16), _f32)],
    mesh=_MESH,
    compiler_params=pltpu.CompilerParams(use_tc_tiling_on_sc=False),
    scratch_types=[
        pltpu.VMEM((CH,), jnp.int32),
        pltpu.VMEM((CH,), jnp.int32),
        pltpu.VMEM((CH, H), _f32),
        pltpu.VMEM((CH, H), _f32),
        pltpu.VMEM((CH, 16), _f32),
        pltpu.SemaphoreType.DMA,
        pltpu.SemaphoreType.DMA,
        pltpu.VMEM_SHARED((NP, H), _f32),
        pltpu.VMEM_SHARED((NP, H), _f32),
        pltpu.VMEM_SHARED((NP, 16), _f32),
        pltpu.VMEM_SHARED((NP, 16), _f32),
    ])


def _sc_conv_body(xr_hbm, xo_hbm, src_hbm, dst_hbm, z64_hbm,
                  mo_out, mr_out,
                  src_v, dst_v, rowr_v, rowo_v, sem1, sem2,
                  mo_sh, mr_sh):
    c = lax.axis_index("c")
    s = lax.axis_index("s")
    wid = c * NS + s
    r0 = s * RPT
    pltpu.sync_copy(z64_hbm.at[pl.ds(r0, RPT)], mo_sh.at[pl.ds(r0, RPT)])
    pltpu.sync_copy(z64_hbm.at[pl.ds(r0, RPT)], mr_sh.at[pl.ds(r0, RPT)])
    plsc.subcore_barrier()

    def chunk(i, carry):
        base = wid * EPW + i * CH
        pltpu.sync_copy(src_hbm.at[pl.ds(base, CH)], src_v)
        pltpu.sync_copy(dst_hbm.at[pl.ds(base, CH)], dst_v)
        pltpu.async_copy(xr_hbm.at[src_v], rowr_v, sem1).wait()
        pltpu.async_copy(xo_hbm.at[dst_v], rowo_v, sem2).wait()
        pltpu.sync_copy(rowr_v, mo_sh.at[dst_v], add=True)
        pltpu.sync_copy(rowo_v, mr_sh.at[src_v], add=True)
        return carry

    lax.fori_loop(0, NCHUNK, chunk, 0)
    plsc.subcore_barrier()
    pltpu.sync_copy(mo_sh.at[pl.ds(r0, RPT)], mo_out.at[c, pl.ds(r0, RPT)])
    pltpu.sync_copy(mr_sh.at[pl.ds(r0, RPT)], mr_out.at[c, pl.ds(r0, RPT)])


_sc_conv = pl.kernel(
    _sc_conv_body,
    out_type=[jax.ShapeDtypeStruct((NC, NP, H), _f32),
              jax.ShapeDtypeStruct((NC, NP, H), _f32)],
    mesh=_MESH,
    compiler_params=pltpu.CompilerParams(use_tc_tiling_on_sc=False),
    scratch_types=[
        pltpu.VMEM((CH,), jnp.int32),
        pltpu.VMEM((CH,), jnp.int32),
        pltpu.VMEM((CH, H), _f32),
        pltpu.VMEM((CH, H), _f32),
        pltpu.SemaphoreType.DMA,
        pltpu.SemaphoreType.DMA,
        pltpu.VMEM_SHARED((NP, H), _f32),
        pltpu.VMEM_SHARED((NP, H), _f32),
    ])


def _sc_score_body(p_hbm, q_hbm, src_hbm, dst_hbm, w2b_hbm, b2b_hbm,
                   out_hbm,
                   src_v, dst_v, pr_v, qr_v, w2_v, b2_v, sco_v, sem1, sem2):
    wid = _worker_id()
    pltpu.sync_copy(w2b_hbm, w2_v)
    pltpu.sync_copy(b2b_hbm, b2_v)

    def chunk(i, carry):
        base = wid * EPW + i * CH
        pltpu.sync_copy(src_hbm.at[pl.ds(base, CH)], src_v)
        pltpu.sync_copy(dst_hbm.at[pl.ds(base, CH)], dst_v)
        pltpu.async_copy(p_hbm.at[src_v], pr_v, sem1).wait()
        pltpu.async_copy(q_hbm.at[dst_v], qr_v, sem2).wait()
        for g in range(CH // 16):
            eids = lax.iota(jnp.int32, 16) + (16 * g)
            acc = b2_v[...]
            for k in range(H):
                kv = jnp.full((16,), k, jnp.int32)
                pv = plsc.load_gather(pr_v, [eids, kv])
                qv = plsc.load_gather(qr_v, [eids, kv])
                acc = acc + jnp.maximum(pv + qv, 0.0) * w2_v[k]
            sco_v[pl.ds(16 * g, 16)] = acc
        pltpu.sync_copy(sco_v, out_hbm.at[pl.ds(base, CH)])
        return carry

    lax.fori_loop(0, NCHUNK, chunk, 0)


_sc_score = pl.kernel(
    _sc_score_body,
    out_type=jax.ShapeDtypeStruct((E,), _f32),
    mesh=_MESH,
    compiler_params=pltpu.CompilerParams(use_tc_tiling_on_sc=False,
                                         needs_layout_passes=False),
    scratch_types=[
        pltpu.VMEM((CH,), jnp.int32),
        pltpu.VMEM((CH,), jnp.int32),
        pltpu.VMEM((CH, H), _f32),
        pltpu.VMEM((CH, H), _f32),
        pltpu.VMEM((H, 16), _f32),
        pltpu.VMEM((16,), _f32),
        pltpu.VMEM((CH,), _f32),
        pltpu.SemaphoreType.DMA,
        pltpu.SemaphoreType.DMA,
    ])


def kernel(rider_x, order_x,
           rmlp_W1, rmlp_b1, rmlp_W2, rmlp_b2,
           omlp_W1, omlp_b1, omlp_W2, omlp_b2,
           c1_ro_Wl, c1_ro_bl, c1_ro_Wr,
           c1_or_Wl, c1_or_bl, c1_or_Wr,
           c2_ro_Wl, c2_ro_bl, c2_ro_Wr,
           c2_or_Wl, c2_or_bl, c2_or_Wr,
           sc_W1, sc_b1, sc_W2, sc_b2,
           edge_index):
    src = edge_index[0].astype(jnp.int32)
    dst = edge_index[1].astype(jnp.int32)
    z64 = jnp.zeros((NP, H), _f32)
    z16 = jnp.zeros((NP, 16), _f32)
    ones16 = jnp.ones((CH, 16), _f32)

    xr, xo = _tc_encode(rider_x, order_x,
                        rmlp_W1, rmlp_b1.reshape(1, H),
                        rmlp_W2, rmlp_b2.reshape(1, H),
                        omlp_W1, omlp_b1.reshape(1, H),
                        omlp_W2, omlp_b2.reshape(1, H))

    mo_s, mr_s, co_s, cr_s = _sc_conv_cnt(xr, xo, src, dst, z64, z16, ones16)

    xo1, xr1 = _tc_conv(mo_s, co_s, xo, c1_ro_Wl, c1_ro_bl.reshape(1, H),
                        c1_ro_Wr,
                        mr_s, cr_s, xr, c1_or_Wl, c1_or_bl.reshape(1, H),
                        c1_or_Wr)

    mo2_s, mr2_s = _sc_conv(xr1, xo1, src, dst, z64)

    q, p = _tc_final(mo2_s, co_s, xo1, c2_ro_Wl, c2_ro_bl.reshape(1, H),
                     c2_ro_Wr, sc_W1[H:], sc_b1.reshape(1, H),
                     mr2_s, cr_s, xr1, c2_or_Wl, c2_or_bl.reshape(1, H),
                     c2_or_Wr, sc_W1[:H])

    w2b = jnp.tile(sc_W2, (1, 16))
    b2b = jnp.broadcast_to(sc_b2.astype(_f32), (16,))
    return _sc_score(p, q, src, dst, w2b, b2b)
